# baseline (speedup 1.0000x reference)
import dataclasses
import functools

import jax
import jax.numpy as jnp
from jax import lax
from jax.experimental import pallas as pl
from jax.experimental.pallas import tpu as pltpu
from jax.experimental.pallas import tpu_sc as plsc

N = 50000
D = 128
E = 100000
NREL = 6
NSUB = 16
NRANGE = 4
RANGE = 12800
N_PAD = RANGE * NRANGE
WIN = 128
EPT = E // NSUB
RW = (EPT + WIN - 1) // WIN
RW_PAD = 56
NWIN = 296
CAP2 = NWIN * 2 * WIN
ROWS_PT = RANGE // NSUB
BN = 5000
NBLK = N // BN

_F32 = jnp.float32
_I32 = jnp.int32


def _gn(x, s, b):
    m = jnp.mean(x, axis=-1, keepdims=True)
    v = jnp.mean((x - m) ** 2, axis=-1, keepdims=True)
    return (x - m) / jnp.sqrt(v + 1e-5) * s + b


def _dot(a, b):
    return jnp.dot(a, b, preferred_element_type=_F32)


def _enc_body(x, w1w, b1w, w2w, sw, bw, w1s, b1s, w2s, ss, bs,
              mw1, mw2, sm, bm, fct, wr, res_o, tb_o, mall_o):
    xx = x[...]
    h = jnp.maximum(xx[:, 0:1] * w1w[0:1, :] + xx[:, 1:2] * w1w[1:2, :]
                    + b1w[...], 0.0)
    fw = _gn(_dot(h, w2w[...]), sw[...], bw[...])
    h = jnp.maximum(xx[:, 2:3] * w1s[0:1, :] + xx[:, 3:4] * w1s[1:2, :]
                    + b1s[...], 0.0)
    fs = _gn(_dot(h, w2s[...]), ss[...], bs[...])
    f = jnp.maximum(fw + fs, 0.0)
    mv = mw2[...]
    y = (_dot(f, mw1[...]) + xx[:, 4:5] * mv[0:1, :] + xx[:, 5:6] * mv[1:2, :]
         + xx[:, 6:7] * mv[2:3, :] + xx[:, 7:8] * mv[3:4, :])
    f = jnp.maximum(_gn(y, sm[...], bm[...]), 0.0)
    res_o[...] = f
    tb_o[...] = _dot(f, fct[...])
    for r in range(NREL):
        mall_o[r] = _dot(f, wr[r])


def _round_core(t, res, fns, fnb, w2, g2s, g2b):
    x = jnp.maximum(_gn(t[...], fns[...], fnb[...]), 0.0)
    f = _gn(_dot(x, w2[...]), g2s[...], g2b[...])
    return jnp.maximum(f + res[...], 0.0)


def _round_body(t, res, fns, fnb, w2, g2s, g2b, fct, wr, res_o, tb_o, mall_o):
    f = _round_core(t, res, fns, fnb, w2, g2s, g2b)
    res_o[...] = f
    tb_o[...] = _dot(f, fct[...])
    for r in range(NREL):
        mall_o[r] = _dot(f, wr[r])


def _final_body(t, res, fns, fnb, w2, g2s, g2b, feat_o):
    feat_o[...] = _round_core(t, res, fns, fnb, w2, g2s, g2b)


def _full(shape):
    rank = len(shape)
    return pl.BlockSpec(shape, lambda n, _r=rank: (0,) * _r)


_ROW = pl.BlockSpec((BN, D), lambda n: (n, 0))
_MALL = pl.BlockSpec((NREL, BN, D), lambda n: (0, n, 0))
_CP = pltpu.CompilerParams(dimension_semantics=("arbitrary",))

_TB_SHAPE = jax.ShapeDtypeStruct((N_PAD, D), _F32)
_MALL_SHAPE = jax.ShapeDtypeStruct((NREL, N, D), _F32)
_ROW_SHAPE = jax.ShapeDtypeStruct((N, D), _F32)


def _enc_call(x8, wts):
    in_specs = [pl.BlockSpec((BN, 8), lambda n: (n, 0))]
    in_specs += [_full(w.shape) for w in wts]
    return pl.pallas_call(
        _enc_body, grid=(NBLK,), in_specs=in_specs,
        out_specs=[_ROW, _ROW, _MALL],
        out_shape=[_ROW_SHAPE, _TB_SHAPE, _MALL_SHAPE],
        compiler_params=_CP,
    )(x8, *wts)


def _round_call(temp, res, wts):
    in_specs = [_ROW, _ROW] + [_full(w.shape) for w in wts]
    return pl.pallas_call(
        _round_body, grid=(NBLK,), in_specs=in_specs,
        out_specs=[_ROW, _ROW, _MALL],
        out_shape=[_ROW_SHAPE, _TB_SHAPE, _MALL_SHAPE],
        compiler_params=_CP,
    )(temp, res, *wts)


def _final_call(temp, res, wts):
    in_specs = [_ROW, _ROW] + [_full(w.shape) for w in wts]
    return pl.pallas_call(
        _final_body, grid=(NBLK,), in_specs=in_specs,
        out_specs=[_ROW], out_shape=[_ROW_SHAPE],
        compiler_params=_CP,
    )(temp, res, *wts)[0]


_MESH = plsc.VectorSubcoreMesh(core_axis_name="c", subcore_axis_name="s")

_SC_CP = pltpu.CompilerParams()
if "needs_layout_passes" in pltpu.CompilerParams.__dataclass_fields__:
    _SC_CP = dataclasses.replace(_SC_CP, needs_layout_passes=False)


def _partition_call(ui, gi):
    @functools.partial(
        pl.kernel,
        out_type=(jax.ShapeDtypeStruct((NSUB, NRANGE, CAP2), _I32),
                  jax.ShapeDtypeStruct((NSUB, 16), _I32)),
        mesh=_MESH,
        scratch_types=[
            pltpu.VMEM((RW_PAD, WIN), _I32),
            pltpu.VMEM((RW_PAD, WIN), _I32),
            pltpu.VMEM((CAP2,), _I32),
            pltpu.VMEM((16,), _I32),
        ],
        compiler_params=_SC_CP,
    )
    def part_k(ui_h, gi_h, lw_h, cnt_h, uin, vin, lw1d, cnt_v):
        core = lax.axis_index("c")
        sub = lax.axis_index("s")

        @pl.when(core == 0)
        def _():
            lanes = lax.iota(_I32, 16)
            cntvec = jnp.zeros((16,), _I32)
            dummy_u = jnp.full((16,), RANGE, _I32)
            dummy_v = jnp.zeros((16,), _I32)
            for p in range(NRANGE):
                lo = p * RANGE
                cursor = jnp.int32(0)
                for r in range(NREL):
                    pltpu.sync_copy(ui_h.at[sub, r], uin)
                    pltpu.sync_copy(gi_h.at[sub, r], vin)

                    def row_body(j, cur, _lo=lo):
                        for k in range(8):
                            uvec = uin[j, pl.ds(k * 16, 16)]
                            vvec = vin[j, pl.ds(k * 16, 16)]
                            msk = (uvec >= _lo) & (uvec < _lo + RANGE)
                            loc = uvec - _lo
                            mi = msk.astype(_I32)
                            inc = plsc.cumsum(mi)
                            q = cur + inc - mi
                            base = (q >> 7) << 8
                            slot = q & (WIN - 1)
                            plsc.store_scatter(
                                lw1d, [base + slot], vvec, mask=msk)
                            plsc.store_scatter(
                                lw1d, [base + WIN + slot], loc, mask=msk)
                            cur = cur + jnp.sum(mi)
                        return cur

                    cursor = lax.fori_loop(0, RW, row_body, cursor)
                for k in range(8):
                    q = cursor + k * 16 + lanes
                    base = (q >> 7) << 8
                    slot = q & (WIN - 1)
                    plsc.store_scatter(lw1d, [base + slot], dummy_v)
                    plsc.store_scatter(lw1d, [base + WIN + slot], dummy_u)
                wc = (cursor + WIN - 1) // WIN
                cntvec = jnp.where(lanes == p, wc, cntvec)
                pltpu.sync_copy(lw1d, lw_h.at[sub, p])
            cnt_v[...] = cntvec
            pltpu.sync_copy(cnt_v, cnt_h.at[sub])

    return part_k(ui, gi)


def _scatter_call(tb, mall, lw, cnt):
    @functools.partial(
        pl.kernel,
        out_type=jax.ShapeDtypeStruct((N_PAD, D), _F32),
        mesh=_MESH,
        scratch_types=[
            pltpu.VMEM((2 * WIN,), _I32),
            pltpu.VMEM((WIN, D), _F32),
            pltpu.VMEM_SHARED((RANGE + 8, D), _F32),
            pltpu.VMEM((16,), _I32),
            pltpu.SemaphoreType.DMA,
        ],
        compiler_params=_SC_CP,
    )
    def scat_k(tb_h, mall_h, lw_h, cnt_h, out_h,
               win0, buf0, acc, cnt_v, sem0):
        core = lax.axis_index("c")
        sub = lax.axis_index("s")
        pltpu.sync_copy(cnt_h.at[sub], cnt_v)
        cvec = cnt_v[...]
        for cc in range(NRANGE // 2):
            p = core * (NRANGE // 2) + cc
            rows_in = pl.ds(p * RANGE + sub * ROWS_PT, ROWS_PT)
            rows_acc = pl.ds(sub * ROWS_PT, ROWS_PT)
            pltpu.sync_copy(tb_h.at[rows_in], acc.at[rows_acc])
            plsc.subcore_barrier()
            nw = jnp.where(core == 0, cvec[cc], cvec[NRANGE // 2 + cc])

            def win_body(i, _):
                off = i * 2 * WIN
                pltpu.sync_copy(lw_h.at[sub, p, pl.ds(off, 2 * WIN)], win0)
                pltpu.async_copy(
                    mall_h.at[win0.at[pl.ds(0, WIN)]], buf0, sem0).wait()
                for k in range(8):
                    uvec = win0[pl.ds(WIN + k * 16, 16)]
                    pltpu.sync_copy(
                        buf0.at[pl.ds(k * 16, 16)], acc.at[uvec], add=True)
                return 0

            lax.fori_loop(0, nw, win_body, 0)
            plsc.subcore_barrier()
            pltpu.sync_copy(acc.at[rows_acc], out_h.at[rows_in])
            plsc.subcore_barrier()

    return scat_k(tb, mall, lw, cnt)


def _pack_edges(us, vs):
    ul, vl = [], []
    for r, (u, v) in enumerate(zip(us, vs)):
        u2 = jnp.concatenate(
            [u.reshape(NSUB, EPT),
             jnp.full((NSUB, RW_PAD * WIN - EPT), N, _I32)], axis=1)
        v2 = jnp.concatenate(
            [(v + r * N).reshape(NSUB, EPT),
             jnp.zeros((NSUB, RW_PAD * WIN - EPT), _I32)], axis=1)
        ul.append(u2.reshape(NSUB, RW_PAD, WIN))
        vl.append(v2.reshape(NSUB, RW_PAD, WIN))
    return jnp.stack(ul, axis=1), jnp.stack(vl, axis=1)


def kernel(ctrs, feats, turn, control, intersect,
           pre0_u, pre0_v, pre1_u, pre1_v, suc0_u, suc0_v, suc1_u, suc1_v,
           left_u, left_v, right_u, right_v, idcs,
           Win_W1, Win_b1, Win_W2, Win_gn_s, Win_gn_b,
           Seg_W1, Seg_b1, Seg_W2, Seg_gn_s, Seg_gn_b,
           Meta_W, Meta_gn_s, Meta_gn_b,
           Fctr, Fpre0, Fpre1, Fsuc0, Fsuc1, Fleft, Fright,
           Fnorm_s, Fnorm_b, Fctr2_W, Fctr2_gn_s, Fctr2_gn_b):
    x8 = jnp.concatenate(
        [ctrs, feats, turn, control[:, None], intersect[:, None]], axis=1)
    ui, gi = _pack_edges(
        [pre0_u, pre1_u, suc0_u, suc1_u, left_u, right_u],
        [pre0_v, pre1_v, suc0_v, suc1_v, left_v, right_v])
    lw, cnt = _partition_call(ui, gi)

    def row2(v):
        return v[None, :]

    rel = jnp.stack([Fpre0, Fpre1, Fsuc0, Fsuc1, Fleft, Fright], axis=1)

    enc_wts = [Win_W1, row2(Win_b1), Win_W2, row2(Win_gn_s), row2(Win_gn_b),
               Seg_W1, row2(Seg_b1), Seg_W2, row2(Seg_gn_s), row2(Seg_gn_b),
               Meta_W[:D], Meta_W[D:], row2(Meta_gn_s), row2(Meta_gn_b),
               Fctr[0], rel[0]]
    res, tb, mall = _enc_call(x8, enc_wts)

    for i in range(4):
        mflat = mall.reshape(NREL * N, D)
        temp = _scatter_call(tb, mflat, lw, cnt)
        gwts = [row2(Fnorm_s[i]), row2(Fnorm_b[i]), Fctr2_W[i],
                row2(Fctr2_gn_s[i]), row2(Fctr2_gn_b[i])]
        if i < 3:
            nwts = gwts + [Fctr[i + 1], rel[i + 1]]
            res, tb, mall = _round_call(temp, res, nwts)
        else:
            feat = _final_call(temp, res, gwts)

    return (feat, idcs, ctrs)


# speedup vs baseline: 1.0166x; 1.0166x over previous
import dataclasses
import functools

import jax
import jax.numpy as jnp
from jax import lax
from jax.experimental import pallas as pl
from jax.experimental.pallas import tpu as pltpu
from jax.experimental.pallas import tpu_sc as plsc

N = 50000
D = 128
E = 100000
NREL = 6
NSUB = 16
NRANGE = 4
RANGE = 12800
N_PAD = RANGE * NRANGE
WIN = 128
EPT = E // NSUB
RW = (EPT + WIN - 1) // WIN
RW_PAD = 56
NWIN = 296
CAP2 = NWIN * 2 * WIN
ROWS_PT = RANGE // NSUB
BN = 2000
NBLK = N // BN

_F32 = jnp.float32
_I32 = jnp.int32


def _gn(x, s, b):
    m = jnp.mean(x, axis=-1, keepdims=True)
    v = jnp.mean((x - m) ** 2, axis=-1, keepdims=True)
    return (x - m) / jnp.sqrt(v + 1e-5) * s + b


def _dot(a, b):
    return jnp.dot(a, b, preferred_element_type=_F32)


def _enc_body(x, w1w, b1w, w2w, sw, bw, w1s, b1s, w2s, ss, bs,
              mw1, mw2, sm, bm, fct, wr, res_o, tb_o, mall_o):
    xx = x[...]
    h = jnp.maximum(xx[:, 0:1] * w1w[0:1, :] + xx[:, 1:2] * w1w[1:2, :]
                    + b1w[...], 0.0)
    fw = _gn(_dot(h, w2w[...]), sw[...], bw[...])
    h = jnp.maximum(xx[:, 2:3] * w1s[0:1, :] + xx[:, 3:4] * w1s[1:2, :]
                    + b1s[...], 0.0)
    fs = _gn(_dot(h, w2s[...]), ss[...], bs[...])
    f = jnp.maximum(fw + fs, 0.0)
    mv = mw2[...]
    y = (_dot(f, mw1[...]) + xx[:, 4:5] * mv[0:1, :] + xx[:, 5:6] * mv[1:2, :]
         + xx[:, 6:7] * mv[2:3, :] + xx[:, 7:8] * mv[3:4, :])
    f = jnp.maximum(_gn(y, sm[...], bm[...]), 0.0)
    res_o[...] = f
    tb_o[...] = _dot(f, fct[...])
    for r in range(NREL):
        mall_o[r] = _dot(f, wr[r])


def _round_core(t, res, fns, fnb, w2, g2s, g2b):
    x = jnp.maximum(_gn(t[...], fns[...], fnb[...]), 0.0)
    f = _gn(_dot(x, w2[...]), g2s[...], g2b[...])
    return jnp.maximum(f + res[...], 0.0)


def _round_body(t, res, fns, fnb, w2, g2s, g2b, fct, wr, res_o, tb_o, mall_o):
    f = _round_core(t, res, fns, fnb, w2, g2s, g2b)
    res_o[...] = f
    tb_o[...] = _dot(f, fct[...])
    for r in range(NREL):
        mall_o[r] = _dot(f, wr[r])


def _final_body(t, res, fns, fnb, w2, g2s, g2b, feat_o):
    feat_o[...] = _round_core(t, res, fns, fnb, w2, g2s, g2b)


def _full(shape):
    rank = len(shape)
    return pl.BlockSpec(shape, lambda n, _r=rank: (0,) * _r)


_ROW = pl.BlockSpec((BN, D), lambda n: (n, 0))
_MALL = pl.BlockSpec((NREL, BN, D), lambda n: (0, n, 0))
_CP = pltpu.CompilerParams(dimension_semantics=("arbitrary",))

_TB_SHAPE = jax.ShapeDtypeStruct((N_PAD, D), _F32)
_MALL_SHAPE = jax.ShapeDtypeStruct((NREL, N, D), _F32)
_ROW_SHAPE = jax.ShapeDtypeStruct((N, D), _F32)


def _enc_call(x8, wts):
    in_specs = [pl.BlockSpec((BN, 8), lambda n: (n, 0))]
    in_specs += [_full(w.shape) for w in wts]
    return pl.pallas_call(
        _enc_body, grid=(NBLK,), in_specs=in_specs,
        out_specs=[_ROW, _ROW, _MALL],
        out_shape=[_ROW_SHAPE, _TB_SHAPE, _MALL_SHAPE],
        compiler_params=_CP,
    )(x8, *wts)


def _round_call(temp, res, wts):
    in_specs = [_ROW, _ROW] + [_full(w.shape) for w in wts]
    return pl.pallas_call(
        _round_body, grid=(NBLK,), in_specs=in_specs,
        out_specs=[_ROW, _ROW, _MALL],
        out_shape=[_ROW_SHAPE, _TB_SHAPE, _MALL_SHAPE],
        compiler_params=_CP,
    )(temp, res, *wts)


def _final_call(temp, res, wts):
    in_specs = [_ROW, _ROW] + [_full(w.shape) for w in wts]
    return pl.pallas_call(
        _final_body, grid=(NBLK,), in_specs=in_specs,
        out_specs=[_ROW], out_shape=[_ROW_SHAPE],
        compiler_params=_CP,
    )(temp, res, *wts)[0]


_MESH = plsc.VectorSubcoreMesh(core_axis_name="c", subcore_axis_name="s")

_SC_CP = pltpu.CompilerParams()
if "needs_layout_passes" in pltpu.CompilerParams.__dataclass_fields__:
    _SC_CP = dataclasses.replace(_SC_CP, needs_layout_passes=False)


def _partition_call(ui, gi):
    @functools.partial(
        pl.kernel,
        out_type=(jax.ShapeDtypeStruct((NSUB, NRANGE, CAP2), _I32),
                  jax.ShapeDtypeStruct((NSUB, 16), _I32)),
        mesh=_MESH,
        scratch_types=[
            pltpu.VMEM((RW_PAD, WIN), _I32),
            pltpu.VMEM((RW_PAD, WIN), _I32),
            pltpu.VMEM((CAP2,), _I32),
            pltpu.VMEM((16,), _I32),
        ],
        compiler_params=_SC_CP,
    )
    def part_k(ui_h, gi_h, lw_h, cnt_h, uin, vin, lw1d, cnt_v):
        core = lax.axis_index("c")
        sub = lax.axis_index("s")

        @pl.when(core == 0)
        def _():
            lanes = lax.iota(_I32, 16)
            cntvec = jnp.zeros((16,), _I32)
            dummy_u = jnp.full((16,), RANGE, _I32)
            dummy_v = jnp.zeros((16,), _I32)
            for p in range(NRANGE):
                lo = p * RANGE
                cursor = jnp.int32(0)
                for r in range(NREL):
                    pltpu.sync_copy(ui_h.at[sub, r], uin)
                    pltpu.sync_copy(gi_h.at[sub, r], vin)

                    def row_body(j, cur, _lo=lo):
                        for k in range(8):
                            uvec = uin[j, pl.ds(k * 16, 16)]
                            vvec = vin[j, pl.ds(k * 16, 16)]
                            msk = (uvec >= _lo) & (uvec < _lo + RANGE)
                            loc = uvec - _lo
                            mi = msk.astype(_I32)
                            inc = plsc.cumsum(mi)
                            q = cur + inc - mi
                            base = (q >> 7) << 8
                            slot = q & (WIN - 1)
                            plsc.store_scatter(
                                lw1d, [base + slot], vvec, mask=msk)
                            plsc.store_scatter(
                                lw1d, [base + WIN + slot], loc, mask=msk)
                            cur = cur + jnp.sum(mi)
                        return cur

                    cursor = lax.fori_loop(0, RW, row_body, cursor)
                for k in range(8):
                    q = cursor + k * 16 + lanes
                    base = (q >> 7) << 8
                    slot = q & (WIN - 1)
                    plsc.store_scatter(lw1d, [base + slot], dummy_v)
                    plsc.store_scatter(lw1d, [base + WIN + slot], dummy_u)
                wc = (cursor + WIN - 1) // WIN
                cntvec = jnp.where(lanes == p, wc, cntvec)
                pltpu.sync_copy(lw1d, lw_h.at[sub, p])
            cnt_v[...] = cntvec
            pltpu.sync_copy(cnt_v, cnt_h.at[sub])

    return part_k(ui, gi)


def _scatter_call(tb, mall, lw, cnt):
    @functools.partial(
        pl.kernel,
        out_type=jax.ShapeDtypeStruct((N_PAD, D), _F32),
        mesh=_MESH,
        scratch_types=[
            pltpu.VMEM((2 * WIN,), _I32),
            pltpu.VMEM((WIN, D), _F32),
            pltpu.VMEM_SHARED((RANGE + 8, D), _F32),
            pltpu.VMEM((16,), _I32),
            pltpu.SemaphoreType.DMA,
        ],
        compiler_params=_SC_CP,
    )
    def scat_k(tb_h, mall_h, lw_h, cnt_h, out_h,
               win0, buf0, acc, cnt_v, sem0):
        core = lax.axis_index("c")
        sub = lax.axis_index("s")
        pltpu.sync_copy(cnt_h.at[sub], cnt_v)
        cvec = cnt_v[...]
        for cc in range(NRANGE // 2):
            p = core * (NRANGE // 2) + cc
            rows_in = pl.ds(p * RANGE + sub * ROWS_PT, ROWS_PT)
            rows_acc = pl.ds(sub * ROWS_PT, ROWS_PT)
            pltpu.sync_copy(tb_h.at[rows_in], acc.at[rows_acc])
            plsc.subcore_barrier()
            nw = jnp.where(core == 0, cvec[cc], cvec[NRANGE // 2 + cc])

            def win_body(i, _):
                off = i * 2 * WIN
                pltpu.sync_copy(lw_h.at[sub, p, pl.ds(off, 2 * WIN)], win0)
                pltpu.async_copy(
                    mall_h.at[win0.at[pl.ds(0, WIN)]], buf0, sem0).wait()
                for k in range(8):
                    uvec = win0[pl.ds(WIN + k * 16, 16)]
                    pltpu.sync_copy(
                        buf0.at[pl.ds(k * 16, 16)], acc.at[uvec], add=True)
                return 0

            lax.fori_loop(0, nw, win_body, 0)
            plsc.subcore_barrier()
            pltpu.sync_copy(acc.at[rows_acc], out_h.at[rows_in])
            plsc.subcore_barrier()

    return scat_k(tb, mall, lw, cnt)


def _pack_edges(us, vs):
    ul, vl = [], []
    for r, (u, v) in enumerate(zip(us, vs)):
        u2 = jnp.concatenate(
            [u.reshape(NSUB, EPT),
             jnp.full((NSUB, RW_PAD * WIN - EPT), N, _I32)], axis=1)
        v2 = jnp.concatenate(
            [(v + r * N).reshape(NSUB, EPT),
             jnp.zeros((NSUB, RW_PAD * WIN - EPT), _I32)], axis=1)
        ul.append(u2.reshape(NSUB, RW_PAD, WIN))
        vl.append(v2.reshape(NSUB, RW_PAD, WIN))
    return jnp.stack(ul, axis=1), jnp.stack(vl, axis=1)


def kernel(ctrs, feats, turn, control, intersect,
           pre0_u, pre0_v, pre1_u, pre1_v, suc0_u, suc0_v, suc1_u, suc1_v,
           left_u, left_v, right_u, right_v, idcs,
           Win_W1, Win_b1, Win_W2, Win_gn_s, Win_gn_b,
           Seg_W1, Seg_b1, Seg_W2, Seg_gn_s, Seg_gn_b,
           Meta_W, Meta_gn_s, Meta_gn_b,
           Fctr, Fpre0, Fpre1, Fsuc0, Fsuc1, Fleft, Fright,
           Fnorm_s, Fnorm_b, Fctr2_W, Fctr2_gn_s, Fctr2_gn_b):
    x8 = jnp.concatenate(
        [ctrs, feats, turn, control[:, None], intersect[:, None]], axis=1)
    ui, gi = _pack_edges(
        [pre0_u, pre1_u, suc0_u, suc1_u, left_u, right_u],
        [pre0_v, pre1_v, suc0_v, suc1_v, left_v, right_v])
    lw, cnt = _partition_call(ui, gi)

    def row2(v):
        return v[None, :]

    rel = jnp.stack([Fpre0, Fpre1, Fsuc0, Fsuc1, Fleft, Fright], axis=1)

    enc_wts = [Win_W1, row2(Win_b1), Win_W2, row2(Win_gn_s), row2(Win_gn_b),
               Seg_W1, row2(Seg_b1), Seg_W2, row2(Seg_gn_s), row2(Seg_gn_b),
               Meta_W[:D], Meta_W[D:], row2(Meta_gn_s), row2(Meta_gn_b),
               Fctr[0], rel[0]]
    res, tb, mall = _enc_call(x8, enc_wts)

    for i in range(4):
        mflat = mall.reshape(NREL * N, D)
        temp = _scatter_call(tb, mflat, lw, cnt)
        gwts = [row2(Fnorm_s[i]), row2(Fnorm_b[i]), Fctr2_W[i],
                row2(Fctr2_gn_s[i]), row2(Fctr2_gn_b[i])]
        if i < 3:
            nwts = gwts + [Fctr[i + 1], rel[i + 1]]
            res, tb, mall = _round_call(temp, res, nwts)
        else:
            feat = _final_call(temp, res, gwts)

    return (feat, idcs, ctrs)
